# baseline (speedup 1.0000x reference)
import functools

import jax
import jax.numpy as jnp
from jax import lax
from jax.experimental import pallas as pl
from jax.experimental.pallas import tpu as pltpu
from jax.experimental.pallas import tpu_sc as plsc

N = 10000
E = 320000
D = 128

NC = 2
NS = 16
NW = NC * NS
EPW = E // NW
C = 96
NCHUNK = EPW // C
TAIL = EPW - NCHUNK * C
RPT = 624
ZC = 104
DL = D // 16


def _sc_segment_sum(h_hbm, src_hbm, dst_hbm, w_hbm, out_hbm,
                    src_v0, src_v1, dst_v0, dst_v1, w_v0, w_v1,
                    rows_v0, rows_v1, stage_v, tsrc_v, tdst_v, tw_v,
                    acc_sh, semI0, semI1, semG0, semG1):
    cid = lax.axis_index("c")
    sid = lax.axis_index("s")
    wid = cid * NS + sid
    e0 = wid * EPW

    src_v = (src_v0, src_v1)
    dst_v = (dst_v0, dst_v1)
    w_v = (w_v0, w_v1)
    rows_v = (rows_v0, rows_v1)
    semI = (semI0, semI1)
    semG = (semG0, semG1)

    zeros16 = jnp.zeros((16,), jnp.float32)

    def zero_row(j, carry):
        for k in range(DL):
            stage_v[j, pl.ds(k * 16, 16)] = zeros16
        return carry
    lax.fori_loop(0, ZC, zero_row, 0)

    r_lo = sid * RPT

    def zero_acc(t, carry):
        pltpu.sync_copy(stage_v, acc_sh.at[pl.ds(r_lo + t * ZC, ZC)])
        return carry
    lax.fori_loop(0, RPT // ZC, zero_acc, 0)

    @pl.when(sid == NS - 1)
    def _():
        pltpu.sync_copy(stage_v.at[pl.ds(0, 16)], acc_sh.at[pl.ds(N - 16, 16)])
    plsc.subcore_barrier()

    def start_idx(i, b):
        base = e0 + i * C
        pltpu.async_copy(src_hbm.at[pl.ds(base, C)], src_v[b], semI[b])
        pltpu.async_copy(dst_hbm.at[pl.ds(base, C)], dst_v[b], semI[b])
        pltpu.async_copy(w_hbm.at[pl.ds(base, C)], w_v[b].at[pl.ds(0, C)],
                         semI[b])

    def wait_idx(b):
        pltpu.make_async_copy(src_hbm.at[pl.ds(0, C)], src_v[b], semI[b]).wait()
        pltpu.make_async_copy(dst_hbm.at[pl.ds(0, C)], dst_v[b], semI[b]).wait()
        pltpu.make_async_copy(w_hbm.at[pl.ds(0, C)], w_v[b].at[pl.ds(0, C)],
                              semI[b]).wait()

    def start_gather(b):
        pltpu.async_copy(h_hbm.at[src_v[b]], rows_v[b], semG[b])

    def wait_gather(b):
        pltpu.make_async_copy(h_hbm.at[src_v[b]], rows_v[b], semG[b]).wait()

    def scale(b, nrows16):
        @plsc.parallel_loop(0, nrows16 * 16, step=16)
        def group(g):
            w16 = w_v[b][pl.ds(g, 16)]
            for j2 in range(16):
                wj = w16[j2]
                for k in range(DL):
                    sl = pl.ds(k * 16, 16)
                    rows_v[b][g + j2, sl] = rows_v[b][g + j2, sl] * wj

    def scatter(b):
        pltpu.sync_copy(rows_v[b], acc_sh.at[dst_v[b]], add=True)

    start_idx(0, 0)
    start_idx(1, 1)
    wait_idx(0)
    start_gather(0)

    def pipe_iter(g, carry):
        for b in (0, 1):
            ob = 1 - b
            i = 2 * g + b
            wait_idx(ob)
            @pl.when(i + 1 < NCHUNK)
            def _():
                start_gather(ob)
            wait_gather(b)
            scale(b, C // 16)
            scatter(b)
            i2 = jnp.minimum(i + 2, NCHUNK - 1)
            start_idx(i2, b)
        return carry
    lax.fori_loop(0, NCHUNK // 2, pipe_iter, 0)

    wait_idx(1)

    tbase = e0 + NCHUNK * C
    pltpu.sync_copy(src_hbm.at[pl.ds(tbase, TAIL)], tsrc_v)
    pltpu.sync_copy(dst_hbm.at[pl.ds(tbase, TAIL)], tdst_v)
    pltpu.sync_copy(w_hbm.at[pl.ds(tbase, TAIL)], tw_v.at[pl.ds(0, TAIL)])
    pltpu.async_copy(h_hbm.at[tsrc_v], rows_v0.at[pl.ds(0, TAIL)], semG0)
    pltpu.make_async_copy(h_hbm.at[tsrc_v], rows_v0.at[pl.ds(0, TAIL)],
                          semG0).wait()
    w16 = tw_v[pl.ds(0, 16)]
    for j2 in range(TAIL):
        wj = w16[j2]
        for k in range(DL):
            sl = pl.ds(k * 16, 16)
            rows_v0[j2, sl] = rows_v0[j2, sl] * wj
    pltpu.sync_copy(rows_v0.at[pl.ds(0, TAIL)], acc_sh.at[tdst_v], add=True)

    plsc.subcore_barrier()

    pltpu.sync_copy(acc_sh.at[pl.ds(r_lo, RPT)],
                    out_hbm.at[pl.ds(cid * N + r_lo, RPT)])

    @pl.when(sid == NS - 1)
    def _():
        pltpu.sync_copy(acc_sh.at[pl.ds(N - 16, 16)],
                        out_hbm.at[pl.ds(cid * N + N - 16, 16)])


_sc_call = pl.kernel(
    _sc_segment_sum,
    out_type=jax.ShapeDtypeStruct((NC * N, D), jnp.float32),
    mesh=plsc.VectorSubcoreMesh(core_axis_name="c", subcore_axis_name="s"),
    scratch_types=[
        pltpu.VMEM((C,), jnp.int32),
        pltpu.VMEM((C,), jnp.int32),
        pltpu.VMEM((C,), jnp.int32),
        pltpu.VMEM((C,), jnp.int32),
        pltpu.VMEM((C + 16,), jnp.float32),
        pltpu.VMEM((C + 16,), jnp.float32),
        pltpu.VMEM((C, D), jnp.float32),
        pltpu.VMEM((C, D), jnp.float32),
        pltpu.VMEM((ZC, D), jnp.float32),
        pltpu.VMEM((TAIL,), jnp.int32),
        pltpu.VMEM((TAIL,), jnp.int32),
        pltpu.VMEM((TAIL + 16,), jnp.float32),
        pltpu.VMEM_SHARED((N, D), jnp.float32),
        pltpu.SemaphoreType.DMA,
        pltpu.SemaphoreType.DMA,
        pltpu.SemaphoreType.DMA,
        pltpu.SemaphoreType.DMA,
    ],
)


BM = 2000


def _mm_body(p_ref, w_ref, b_ref, o_ref):
    x = p_ref[0] + p_ref[1]
    y = jnp.dot(x, w_ref[...], preferred_element_type=jnp.float32,
                precision=lax.Precision.HIGHEST)
    o_ref[...] = jnp.maximum(y + b_ref[...], 0.0)


_mm_call = pl.pallas_call(
    _mm_body,
    out_shape=jax.ShapeDtypeStruct((N, D), jnp.float32),
    grid=(N // BM,),
    in_specs=[
        pl.BlockSpec((2, BM, D), lambda i: (0, i, 0)),
        pl.BlockSpec((D, D), lambda i: (0, 0)),
        pl.BlockSpec((1, D), lambda i: (0, 0)),
    ],
    out_specs=pl.BlockSpec((BM, D), lambda i: (i, 0)),
)


def kernel(h, edge_index, sym_norm, W, b):
    src = edge_index[0]
    dst = edge_index[1]
    partial = _sc_call(h, src, dst, sym_norm)
    p = partial.reshape(NC, N, D)
    return _mm_call(p, W, b.reshape(1, D))


# speedup vs baseline: 1.0590x; 1.0590x over previous
import functools

import jax
import jax.numpy as jnp
from jax import lax
from jax.experimental import pallas as pl
from jax.experimental.pallas import tpu as pltpu
from jax.experimental.pallas import tpu_sc as plsc

N = 10000
E = 320000
D = 128

NC = 2
NS = 16
NW = NC * NS
EPW = E // NW
C = 128
NCHUNK = EPW // C
TAIL = EPW - NCHUNK * C
RPT = 624
ZC = 104
DL = D // 16


def _sc_segment_sum(h_hbm, src_hbm, dst_hbm, w_hbm, out_hbm,
                    src_v0, src_v1, dst_v0, dst_v1, w_v0, w_v1,
                    rows_v0, rows_v1, stage_v, tsrc_v, tdst_v, tw_v,
                    acc_sh, semI0, semI1, semG0, semG1):
    cid = lax.axis_index("c")
    sid = lax.axis_index("s")
    wid = cid * NS + sid
    e0 = wid * EPW

    src_v = (src_v0, src_v1)
    dst_v = (dst_v0, dst_v1)
    w_v = (w_v0, w_v1)
    rows_v = (rows_v0, rows_v1)
    semI = (semI0, semI1)
    semG = (semG0, semG1)

    zeros16 = jnp.zeros((16,), jnp.float32)

    def zero_row(j, carry):
        for k in range(DL):
            stage_v[j, pl.ds(k * 16, 16)] = zeros16
        return carry
    lax.fori_loop(0, ZC, zero_row, 0)

    r_lo = sid * RPT

    def zero_acc(t, carry):
        pltpu.sync_copy(stage_v, acc_sh.at[pl.ds(r_lo + t * ZC, ZC)])
        return carry
    lax.fori_loop(0, RPT // ZC, zero_acc, 0)

    @pl.when(sid == NS - 1)
    def _():
        pltpu.sync_copy(stage_v.at[pl.ds(0, 16)], acc_sh.at[pl.ds(N - 16, 16)])
    plsc.subcore_barrier()

    def start_idx(i, b):
        base = e0 + i * C
        pltpu.async_copy(src_hbm.at[pl.ds(base, C)], src_v[b], semI[b])
        pltpu.async_copy(dst_hbm.at[pl.ds(base, C)], dst_v[b], semI[b])
        pltpu.async_copy(w_hbm.at[pl.ds(base, C)], w_v[b].at[pl.ds(0, C)],
                         semI[b])

    def wait_idx(b):
        pltpu.make_async_copy(src_hbm.at[pl.ds(0, C)], src_v[b], semI[b]).wait()
        pltpu.make_async_copy(dst_hbm.at[pl.ds(0, C)], dst_v[b], semI[b]).wait()
        pltpu.make_async_copy(w_hbm.at[pl.ds(0, C)], w_v[b].at[pl.ds(0, C)],
                              semI[b]).wait()

    def start_gather(b):
        pltpu.async_copy(h_hbm.at[src_v[b]], rows_v[b], semG[b])

    def wait_gather(b):
        pltpu.make_async_copy(h_hbm.at[src_v[b]], rows_v[b], semG[b]).wait()

    def scale(b, nrows16):
        @plsc.parallel_loop(0, nrows16 * 16, step=16)
        def group(g):
            w16 = w_v[b][pl.ds(g, 16)]
            for j2 in range(16):
                wj = w16[j2]
                for k in range(DL):
                    sl = pl.ds(k * 16, 16)
                    rows_v[b][g + j2, sl] = rows_v[b][g + j2, sl] * wj

    def scatter(b):
        pltpu.sync_copy(rows_v[b], acc_sh.at[dst_v[b]], add=True)

    start_idx(0, 0)
    start_idx(1, 1)
    wait_idx(0)
    start_gather(0)

    def pipe_iter(g, carry):
        for b in (0, 1):
            ob = 1 - b
            i = 2 * g + b
            wait_idx(ob)
            @pl.when(i + 1 < NCHUNK)
            def _():
                start_gather(ob)
            wait_gather(b)
            scale(b, C // 16)
            scatter(b)
            i2 = jnp.minimum(i + 2, NCHUNK - 1)
            start_idx(i2, b)
        return carry
    lax.fori_loop(0, NCHUNK // 2, pipe_iter, 0)

    wait_idx(1)

    tbase = e0 + NCHUNK * C
    pltpu.sync_copy(src_hbm.at[pl.ds(tbase, TAIL)], tsrc_v)
    pltpu.sync_copy(dst_hbm.at[pl.ds(tbase, TAIL)], tdst_v)
    pltpu.sync_copy(w_hbm.at[pl.ds(tbase, TAIL)], tw_v.at[pl.ds(0, TAIL)])
    pltpu.async_copy(h_hbm.at[tsrc_v], rows_v0.at[pl.ds(0, TAIL)], semG0)
    pltpu.make_async_copy(h_hbm.at[tsrc_v], rows_v0.at[pl.ds(0, TAIL)],
                          semG0).wait()
    w16 = tw_v[pl.ds(0, 16)]
    for j2 in range(TAIL):
        wj = w16[j2]
        for k in range(DL):
            sl = pl.ds(k * 16, 16)
            rows_v0[j2, sl] = rows_v0[j2, sl] * wj
    pltpu.sync_copy(rows_v0.at[pl.ds(0, TAIL)], acc_sh.at[tdst_v], add=True)

    plsc.subcore_barrier()

    pltpu.sync_copy(acc_sh.at[pl.ds(r_lo, RPT)],
                    out_hbm.at[pl.ds(cid * N + r_lo, RPT)])

    @pl.when(sid == NS - 1)
    def _():
        pltpu.sync_copy(acc_sh.at[pl.ds(N - 16, 16)],
                        out_hbm.at[pl.ds(cid * N + N - 16, 16)])


_sc_call = pl.kernel(
    _sc_segment_sum,
    out_type=jax.ShapeDtypeStruct((NC * N, D), jnp.float32),
    mesh=plsc.VectorSubcoreMesh(core_axis_name="c", subcore_axis_name="s"),
    scratch_types=[
        pltpu.VMEM((C,), jnp.int32),
        pltpu.VMEM((C,), jnp.int32),
        pltpu.VMEM((C,), jnp.int32),
        pltpu.VMEM((C,), jnp.int32),
        pltpu.VMEM((C + 16,), jnp.float32),
        pltpu.VMEM((C + 16,), jnp.float32),
        pltpu.VMEM((C, D), jnp.float32),
        pltpu.VMEM((C, D), jnp.float32),
        pltpu.VMEM((ZC, D), jnp.float32),
        pltpu.VMEM((TAIL,), jnp.int32),
        pltpu.VMEM((TAIL,), jnp.int32),
        pltpu.VMEM((TAIL + 16,), jnp.float32),
        pltpu.VMEM_SHARED((N, D), jnp.float32),
        pltpu.SemaphoreType.DMA,
        pltpu.SemaphoreType.DMA,
        pltpu.SemaphoreType.DMA,
        pltpu.SemaphoreType.DMA,
    ],
)


BM = 2000


def _mm_body(p_ref, w_ref, b_ref, o_ref):
    x = p_ref[0] + p_ref[1]
    y = jnp.dot(x, w_ref[...], preferred_element_type=jnp.float32,
                precision=lax.Precision.HIGHEST)
    o_ref[...] = jnp.maximum(y + b_ref[...], 0.0)


_mm_call = pl.pallas_call(
    _mm_body,
    out_shape=jax.ShapeDtypeStruct((N, D), jnp.float32),
    grid=(N // BM,),
    in_specs=[
        pl.BlockSpec((2, BM, D), lambda i: (0, i, 0)),
        pl.BlockSpec((D, D), lambda i: (0, 0)),
        pl.BlockSpec((1, D), lambda i: (0, 0)),
    ],
    out_specs=pl.BlockSpec((BM, D), lambda i: (i, 0)),
)


def kernel(h, edge_index, sym_norm, W, b):
    src = edge_index[0]
    dst = edge_index[1]
    partial = _sc_call(h, src, dst, sym_norm)
    p = partial.reshape(NC, N, D)
    return _mm_call(p, W, b.reshape(1, D))
